# Initial kernel scaffold; baseline (speedup 1.0000x reference)
import jax
import jax.numpy as jnp
from jax.experimental import pallas as pl


def kernel(hidden_states, W_base, W_ce1, b_ce1, W_ce2, b_ce2, W_ra1, b_ra1, W_ra2, b_ra2):
    raise NotImplementedError("write your pallas kernel here")


# speedup vs baseline: 1.3228x; 1.3228x over previous
import jax
import jax.numpy as jnp
from jax.experimental import pallas as pl
from jax.experimental.pallas import tpu as pltpu

H = 4096
E = 64
N = 8192
TOP_K = 8

_PREC = jax.lax.Precision.DEFAULT


def _main_pass_kernel(h_ref, w_ref, out_ref, colsum_ref):
    i = pl.program_id(0)
    h = h_ref[...]
    out_ref[...] = jax.lax.dot_general(
        h, w_ref[...], (((1,), (1,)), ((), ())),
        preferred_element_type=jnp.float32, precision=_PREC)
    part = jnp.sum(h, axis=0, keepdims=True)

    @pl.when(i == 0)
    def _():
        colsum_ref[...] = part

    @pl.when(i != 0)
    def _():
        colsum_ref[...] += part


def _ce1_kernel(colsum_ref, w_ref, b_ref, out_ref):
    pooled = colsum_ref[...] * (1.0 / N)
    v = jax.lax.dot_general(
        pooled, w_ref[...], (((1,), (1,)), ((), ())),
        preferred_element_type=jnp.float32, precision=_PREC)
    out_ref[...] = jnp.maximum(v + b_ref[...], 0.0)


def _adapt_kernel(h1_ref, wc2_ref, bc2_ref, wr1_ref, br1_ref, wr2_ref,
                  br2_ref, out_ref):
    context = jax.lax.dot_general(
        h1_ref[...], wc2_ref[...], (((1,), (1,)), ((), ())),
        preferred_element_type=jnp.float32, precision=_PREC) + bc2_ref[...]
    a1 = jnp.tanh(jax.lax.dot_general(
        context, wr1_ref[...], (((1,), (1,)), ((), ())),
        preferred_element_type=jnp.float32, precision=_PREC) + br1_ref[...])
    out_ref[...] = jax.lax.dot_general(
        a1, wr2_ref[...], (((1,), (1,)), ((), ())),
        preferred_element_type=jnp.float32, precision=_PREC) + br2_ref[...]


def _epilogue_kernel(base_ref, adapt_ref, out_ref, idx_ref, wts_ref,
                     var_ref, ent_ref, load_acc, ent_acc):
    i = pl.program_id(0)
    nb = pl.num_programs(0)
    x = base_ref[...] + adapt_ref[...]
    out_ref[...] = x

    m = jnp.max(x, axis=-1, keepdims=True)
    ex = jnp.exp(x - m)
    s = jnp.sum(ex, axis=-1, keepdims=True)
    probs = ex / s
    row_ent = -jnp.sum(probs * jnp.log(probs + 1e-8), axis=-1)
    ent_part = jnp.sum(row_ent)
    load_part = jnp.sum(probs, axis=0, keepdims=True)

    ent_part2d = ent_part.reshape(1, 1)

    @pl.when(i == 0)
    def _():
        load_acc[...] = load_part
        ent_acc[...] = ent_part2d

    @pl.when(i != 0)
    def _():
        load_acc[...] += load_part
        ent_acc[...] += ent_part2d

    ii = jax.lax.broadcasted_iota(jnp.int32, x.shape, 1)
    vals = []
    idxs = []
    for _k in range(TOP_K):
        mval = jnp.max(x, axis=-1, keepdims=True)
        cand = jnp.where(x == mval, ii, E)
        am = jnp.min(cand, axis=-1, keepdims=True)
        vals.append(mval)
        idxs.append(am)
        x = jnp.where(ii == am, -jnp.inf, x)
    topv = jnp.concatenate(vals, axis=1)
    idx_ref[...] = jnp.concatenate(idxs, axis=1)
    e2 = jnp.exp(topv - topv[:, :1])
    wts_ref[...] = e2 / jnp.sum(e2, axis=-1, keepdims=True)

    @pl.when(i == nb - 1)
    def _():
        el = load_acc[...] * (1.0 / N)
        mu = jnp.mean(el)
        var_ref[...] = (jnp.sum((el - mu) ** 2) * (1.0 / (E - 1))).reshape(1, 1)
        ent_ref[...] = ent_acc[...] * (1.0 / N)


def kernel(hidden_states, W_base, W_ce1, b_ce1, W_ce2, b_ce2, W_ra1, b_ra1,
           W_ra2, b_ra2):
    BLK = 512
    base_logits, colsum = pl.pallas_call(
        _main_pass_kernel,
        grid=(N // BLK,),
        in_specs=[
            pl.BlockSpec((BLK, H), lambda i: (i, 0)),
            pl.BlockSpec((E, H), lambda i: (0, 0)),
        ],
        out_specs=[
            pl.BlockSpec((BLK, E), lambda i: (i, 0)),
            pl.BlockSpec((1, H), lambda i: (0, 0)),
        ],
        out_shape=[
            jax.ShapeDtypeStruct((N, E), jnp.float32),
            jax.ShapeDtypeStruct((1, H), jnp.float32),
        ],
    )(hidden_states, W_base)

    CBLK = 512
    H2 = H // 2
    h1 = pl.pallas_call(
        _ce1_kernel,
        grid=(H2 // CBLK,),
        in_specs=[
            pl.BlockSpec((1, H), lambda i: (0, 0)),
            pl.BlockSpec((CBLK, H), lambda i: (i, 0)),
            pl.BlockSpec((1, CBLK), lambda i: (0, i)),
        ],
        out_specs=pl.BlockSpec((1, CBLK), lambda i: (0, i)),
        out_shape=jax.ShapeDtypeStruct((1, H2), jnp.float32),
    )(colsum, W_ce1, b_ce1.reshape(1, H2))

    H4 = H // 4
    adaptation = pl.pallas_call(
        _adapt_kernel,
        in_specs=[pl.BlockSpec(memory_space=pltpu.VMEM)] * 7,
        out_specs=pl.BlockSpec(memory_space=pltpu.VMEM),
        out_shape=jax.ShapeDtypeStruct((1, E), jnp.float32),
    )(h1, W_ce2, b_ce2.reshape(1, H4), W_ra1, b_ra1.reshape(1, H),
      W_ra2, b_ra2.reshape(1, E))

    EB = 1024
    adapted, idx, wts, var_out, ent_out = pl.pallas_call(
        _epilogue_kernel,
        grid=(N // EB,),
        in_specs=[
            pl.BlockSpec((EB, E), lambda i: (i, 0)),
            pl.BlockSpec((1, E), lambda i: (0, 0)),
        ],
        out_specs=[
            pl.BlockSpec((EB, E), lambda i: (i, 0)),
            pl.BlockSpec((EB, TOP_K), lambda i: (i, 0)),
            pl.BlockSpec((EB, TOP_K), lambda i: (i, 0)),
            pl.BlockSpec((1, 1), lambda i: (0, 0)),
            pl.BlockSpec((1, 1), lambda i: (0, 0)),
        ],
        out_shape=[
            jax.ShapeDtypeStruct((N, E), jnp.float32),
            jax.ShapeDtypeStruct((N, TOP_K), jnp.int32),
            jax.ShapeDtypeStruct((N, TOP_K), jnp.float32),
            jax.ShapeDtypeStruct((1, 1), jnp.float32),
            jax.ShapeDtypeStruct((1, 1), jnp.float32),
        ],
        scratch_shapes=[
            pltpu.VMEM((1, E), jnp.float32),
            pltpu.VMEM((1, 1), jnp.float32),
        ],
    )(base_logits, adaptation)

    return (adapted, idx, wts, var_out[0, 0], ent_out[0, 0])


# speedup vs baseline: 1.3513x; 1.0216x over previous
import jax
import jax.numpy as jnp
from jax.experimental import pallas as pl
from jax.experimental.pallas import tpu as pltpu

H = 4096
E = 64
N = 8192
TOP_K = 8

_PREC = jax.lax.Precision.DEFAULT

BLK = 512
CE1_BLK = 512
N_CE1 = (H // 2) // CE1_BLK
EB = 1024
N_EPI = N // EB
ADAPT_STEP = N_CE1
EPI0 = N_CE1 + 1


def _main_pass_kernel(h_ref, w_ref, out_ref, colsum_ref):
    i = pl.program_id(0)
    h = h_ref[...]
    out_ref[...] = jax.lax.dot_general(
        h, w_ref[...], (((1,), (1,)), ((), ())),
        preferred_element_type=jnp.float32, precision=_PREC)
    part = jnp.sum(h, axis=0, keepdims=True)

    @pl.when(i == 0)
    def _():
        colsum_ref[...] = part

    @pl.when(i != 0)
    def _():
        colsum_ref[...] += part


def _router_kernel(colsum_ref, wc1_ref, bc1_ref, wc2_ref, bc2_ref, wr1_ref,
                   br1_ref, wr2_ref, br2_ref, base_ref,
                   out_ref, idx_ref, wts_ref, var_ref, ent_ref,
                   h1_scr, adapt_scr, load_acc, ent_acc):
    i = pl.program_id(0)

    @pl.when(i < N_CE1)
    def _():
        pooled = colsum_ref[...] * (1.0 / N)
        v = jnp.maximum(jax.lax.dot_general(
            pooled, wc1_ref[...], (((1,), (1,)), ((), ())),
            preferred_element_type=jnp.float32, precision=_PREC)
            + bc1_ref[...], 0.0)
        for s in range(N_CE1):
            @pl.when(i == s)
            def _():
                h1_scr[s:s + 1, :] = v

    @pl.when(i == ADAPT_STEP)
    def _():
        context = bc2_ref[...]
        for s in range(N_CE1):
            context += jax.lax.dot_general(
                h1_scr[s:s + 1, :],
                wc2_ref[:, s * CE1_BLK:(s + 1) * CE1_BLK],
                (((1,), (1,)), ((), ())),
                preferred_element_type=jnp.float32, precision=_PREC)
        a1 = jnp.tanh(jax.lax.dot_general(
            context, wr1_ref[...], (((1,), (1,)), ((), ())),
            preferred_element_type=jnp.float32, precision=_PREC)
            + br1_ref[...])
        adapt_scr[...] = jax.lax.dot_general(
            a1, wr2_ref[...], (((1,), (1,)), ((), ())),
            preferred_element_type=jnp.float32, precision=_PREC) + br2_ref[...]

    @pl.when(i >= EPI0)
    def _():
        x = base_ref[...] + adapt_scr[...]
        out_ref[...] = x

        m = jnp.max(x, axis=-1, keepdims=True)
        ex = jnp.exp(x - m)
        s = jnp.sum(ex, axis=-1, keepdims=True)
        probs = ex / s
        row_ent = -jnp.sum(probs * jnp.log(probs + 1e-8), axis=-1)
        ent_part = jnp.sum(row_ent).reshape(1, 1)
        load_part = jnp.sum(probs, axis=0, keepdims=True)

        @pl.when(i == EPI0)
        def _():
            load_acc[...] = load_part
            ent_acc[...] = ent_part

        @pl.when(i != EPI0)
        def _():
            load_acc[...] += load_part
            ent_acc[...] += ent_part

        ii = jax.lax.broadcasted_iota(jnp.int32, x.shape, 1)
        vals = []
        idxs = []
        for _k in range(TOP_K):
            mval = jnp.max(x, axis=-1, keepdims=True)
            cand = jnp.where(x == mval, ii, E)
            am = jnp.min(cand, axis=-1, keepdims=True)
            vals.append(mval)
            idxs.append(am)
            x = jnp.where(ii == am, -jnp.inf, x)
        topv = jnp.concatenate(vals, axis=1)
        idx_ref[...] = jnp.concatenate(idxs, axis=1)
        e2 = jnp.exp(topv - topv[:, :1])
        wts_ref[...] = e2 / jnp.sum(e2, axis=-1, keepdims=True)

        @pl.when(i == EPI0 + N_EPI - 1)
        def _():
            el = load_acc[...] * (1.0 / N)
            mu = jnp.mean(el)
            var_ref[...] = (jnp.sum((el - mu) ** 2)
                            * (1.0 / (E - 1))).reshape(1, 1)
            ent_ref[...] = ent_acc[...] * (1.0 / N)


def kernel(hidden_states, W_base, W_ce1, b_ce1, W_ce2, b_ce2, W_ra1, b_ra1,
           W_ra2, b_ra2):
    base_logits, colsum = pl.pallas_call(
        _main_pass_kernel,
        grid=(N // BLK,),
        in_specs=[
            pl.BlockSpec((BLK, H), lambda i: (i, 0)),
            pl.BlockSpec((E, H), lambda i: (0, 0)),
        ],
        out_specs=[
            pl.BlockSpec((BLK, E), lambda i: (i, 0)),
            pl.BlockSpec((1, H), lambda i: (0, 0)),
        ],
        out_shape=[
            jax.ShapeDtypeStruct((N, E), jnp.float32),
            jax.ShapeDtypeStruct((1, H), jnp.float32),
        ],
    )(hidden_states, W_base)

    H2 = H // 2
    H4 = H // 4
    grid = (EPI0 + N_EPI,)

    def ce1_map(i):
        return (jnp.minimum(i, N_CE1 - 1), 0)

    def bce1_map(i):
        return (0, jnp.minimum(i, N_CE1 - 1))

    def epi_map(i):
        return (jnp.clip(i - EPI0, 0, N_EPI - 1), 0)

    const2 = lambda i: (0, 0)

    adapted, idx, wts, var_out, ent_out = pl.pallas_call(
        _router_kernel,
        grid=grid,
        in_specs=[
            pl.BlockSpec((1, H), const2),
            pl.BlockSpec((CE1_BLK, H), ce1_map),
            pl.BlockSpec((1, CE1_BLK), bce1_map),
            pl.BlockSpec((H4, H2), const2),
            pl.BlockSpec((1, H4), const2),
            pl.BlockSpec((H, H4), const2),
            pl.BlockSpec((1, H), const2),
            pl.BlockSpec((E, H), const2),
            pl.BlockSpec((1, E), const2),
            pl.BlockSpec((EB, E), epi_map),
        ],
        out_specs=[
            pl.BlockSpec((EB, E), epi_map),
            pl.BlockSpec((EB, TOP_K), epi_map),
            pl.BlockSpec((EB, TOP_K), epi_map),
            pl.BlockSpec((1, 1), const2),
            pl.BlockSpec((1, 1), const2),
        ],
        out_shape=[
            jax.ShapeDtypeStruct((N, E), jnp.float32),
            jax.ShapeDtypeStruct((N, TOP_K), jnp.int32),
            jax.ShapeDtypeStruct((N, TOP_K), jnp.float32),
            jax.ShapeDtypeStruct((1, 1), jnp.float32),
            jax.ShapeDtypeStruct((1, 1), jnp.float32),
        ],
        scratch_shapes=[
            pltpu.VMEM((N_CE1, CE1_BLK), jnp.float32),
            pltpu.VMEM((1, E), jnp.float32),
            pltpu.VMEM((1, E), jnp.float32),
            pltpu.VMEM((1, 1), jnp.float32),
        ],
    )(colsum, W_ce1, b_ce1.reshape(1, H2), W_ce2, b_ce2.reshape(1, H4),
      W_ra1, b_ra1.reshape(1, H), W_ra2, b_ra2.reshape(1, E), base_logits)

    return (adapted, idx, wts, var_out[0, 0], ent_out[0, 0])


# speedup vs baseline: 1.7339x; 1.2831x over previous
import jax
import jax.numpy as jnp
from jax.experimental import pallas as pl
from jax.experimental.pallas import tpu as pltpu

H = 4096
E = 64
N = 8192
TOP_K = 8

_PREC = jax.lax.Precision.DEFAULT

BLK = 512
CE1_BLK = 512
N_CE1 = (H // 2) // CE1_BLK
EB = 2048
N_EPI = N // EB
ADAPT_STEP = N_CE1
EPI0 = N_CE1 + 1


def _main_pass_kernel(h_ref, w_ref, out_ref, colsum_ref):
    i = pl.program_id(0)
    h = h_ref[...]
    out_ref[...] = jax.lax.dot_general(
        w_ref[...], h, (((1,), (1,)), ((), ())),
        preferred_element_type=jnp.float32, precision=_PREC)
    part = jnp.sum(h, axis=0, keepdims=True)

    @pl.when(i == 0)
    def _():
        colsum_ref[...] = part

    @pl.when(i != 0)
    def _():
        colsum_ref[...] += part


def _router_kernel(colsum_ref, wc1_ref, bc1_ref, wc2_ref, bc2_ref, wr1_ref,
                   br1_ref, wr2_ref, br2_ref, base_ref,
                   out_ref, idx_ref, wts_ref, var_ref, ent_ref,
                   h1_scr, adapt_scr, load_acc, ent_acc):
    i = pl.program_id(0)

    @pl.when(i < N_CE1)
    def _():
        pooled = colsum_ref[...] * (1.0 / N)
        v = jnp.maximum(jax.lax.dot_general(
            pooled, wc1_ref[...], (((1,), (1,)), ((), ())),
            preferred_element_type=jnp.float32, precision=_PREC)
            + bc1_ref[...], 0.0)
        for s in range(N_CE1):
            @pl.when(i == s)
            def _():
                h1_scr[s:s + 1, :] = v

    @pl.when(i == ADAPT_STEP)
    def _():
        context = bc2_ref[...]
        for s in range(N_CE1):
            context += jax.lax.dot_general(
                h1_scr[s:s + 1, :],
                wc2_ref[:, s * CE1_BLK:(s + 1) * CE1_BLK],
                (((1,), (1,)), ((), ())),
                preferred_element_type=jnp.float32, precision=_PREC)
        a1 = jnp.tanh(jax.lax.dot_general(
            context, wr1_ref[...], (((1,), (1,)), ((), ())),
            preferred_element_type=jnp.float32, precision=_PREC)
            + br1_ref[...])
        adapt = jax.lax.dot_general(
            a1, wr2_ref[...], (((1,), (1,)), ((), ())),
            preferred_element_type=jnp.float32, precision=_PREC) + br2_ref[...]
        adapt_scr[...] = adapt.reshape(E, 1)

    @pl.when(i >= EPI0)
    def _():
        x = base_ref[...] + adapt_scr[...]
        out_ref[...] = x.T

        m = jnp.max(x, axis=0, keepdims=True)
        ex = jnp.exp(x - m)
        s = jnp.sum(ex, axis=0, keepdims=True)
        probs = ex / s
        row_ent = -jnp.sum(probs * jnp.log(probs + 1e-8), axis=0)
        ent_part = jnp.sum(row_ent).reshape(1, 1)
        load_part = jnp.sum(probs, axis=1, keepdims=True)

        @pl.when(i == EPI0)
        def _():
            load_acc[...] = load_part
            ent_acc[...] = ent_part

        @pl.when(i != EPI0)
        def _():
            load_acc[...] += load_part
            ent_acc[...] += ent_part

        ii = jax.lax.broadcasted_iota(jnp.int32, x.shape, 0)
        vals = []
        idxs = []
        for _k in range(TOP_K):
            mval = jnp.max(x, axis=0, keepdims=True)
            cand = jnp.where(x == mval, ii, E)
            am = jnp.min(cand, axis=0, keepdims=True)
            vals.append(mval)
            idxs.append(am)
            x = jnp.where(ii == am, -jnp.inf, x)
        topv = jnp.concatenate(vals, axis=0)
        topi = jnp.concatenate(idxs, axis=0)
        idx_ref[...] = topi.T
        e2 = jnp.exp(topv - topv[:1, :])
        wts_ref[...] = (e2 / jnp.sum(e2, axis=0, keepdims=True)).T

        @pl.when(i == EPI0 + N_EPI - 1)
        def _():
            el = load_acc[...] * (1.0 / N)
            mu = jnp.mean(el)
            var_ref[...] = (jnp.sum((el - mu) ** 2)
                            * (1.0 / (E - 1))).reshape(1, 1)
            ent_ref[...] = ent_acc[...] * (1.0 / N)


def kernel(hidden_states, W_base, W_ce1, b_ce1, W_ce2, b_ce2, W_ra1, b_ra1,
           W_ra2, b_ra2):
    base_logits_t, colsum = pl.pallas_call(
        _main_pass_kernel,
        grid=(N // BLK,),
        in_specs=[
            pl.BlockSpec((BLK, H), lambda i: (i, 0)),
            pl.BlockSpec((E, H), lambda i: (0, 0)),
        ],
        out_specs=[
            pl.BlockSpec((E, BLK), lambda i: (0, i)),
            pl.BlockSpec((1, H), lambda i: (0, 0)),
        ],
        out_shape=[
            jax.ShapeDtypeStruct((E, N), jnp.float32),
            jax.ShapeDtypeStruct((1, H), jnp.float32),
        ],
    )(hidden_states, W_base)

    H2 = H // 2
    H4 = H // 4
    grid = (EPI0 + N_EPI,)

    def ce1_map(i):
        return (jnp.minimum(i, N_CE1 - 1), 0)

    def bce1_map(i):
        return (0, jnp.minimum(i, N_CE1 - 1))

    def epi_map(i):
        return (0, jnp.clip(i - EPI0, 0, N_EPI - 1))

    def epi_map_t(i):
        return (jnp.clip(i - EPI0, 0, N_EPI - 1), 0)

    const2 = lambda i: (0, 0)

    adapted, idx, wts, var_out, ent_out = pl.pallas_call(
        _router_kernel,
        grid=grid,
        in_specs=[
            pl.BlockSpec((1, H), const2),
            pl.BlockSpec((CE1_BLK, H), ce1_map),
            pl.BlockSpec((1, CE1_BLK), bce1_map),
            pl.BlockSpec((H4, H2), const2),
            pl.BlockSpec((1, H4), const2),
            pl.BlockSpec((H, H4), const2),
            pl.BlockSpec((1, H), const2),
            pl.BlockSpec((E, H), const2),
            pl.BlockSpec((1, E), const2),
            pl.BlockSpec((E, EB), epi_map),
        ],
        out_specs=[
            pl.BlockSpec((EB, E), epi_map_t),
            pl.BlockSpec((EB, TOP_K), epi_map_t),
            pl.BlockSpec((EB, TOP_K), epi_map_t),
            pl.BlockSpec((1, 1), const2),
            pl.BlockSpec((1, 1), const2),
        ],
        out_shape=[
            jax.ShapeDtypeStruct((N, E), jnp.float32),
            jax.ShapeDtypeStruct((N, TOP_K), jnp.int32),
            jax.ShapeDtypeStruct((N, TOP_K), jnp.float32),
            jax.ShapeDtypeStruct((1, 1), jnp.float32),
            jax.ShapeDtypeStruct((1, 1), jnp.float32),
        ],
        scratch_shapes=[
            pltpu.VMEM((N_CE1, CE1_BLK), jnp.float32),
            pltpu.VMEM((E, 1), jnp.float32),
            pltpu.VMEM((E, 1), jnp.float32),
            pltpu.VMEM((1, 1), jnp.float32),
        ],
    )(colsum, W_ce1, b_ce1.reshape(1, H2), W_ce2, b_ce2.reshape(1, H4),
      W_ra1, b_ra1.reshape(1, H), W_ra2, b_ra2.reshape(1, E), base_logits_t)

    return (adapted, idx, wts, var_out[0, 0], ent_out[0, 0])
